# baseline (speedup 1.0000x reference)
import jax
import jax.numpy as jnp
from jax.experimental import pallas as pl
from jax.experimental.pallas import tpu as pltpu

_SLOT_ROWS = 1152
_NBUF = 3
_CHUNKS = [1152] * 14 + [256]
_STARTS = [sum(_CHUNKS[:i]) for i in range(len(_CHUNKS))]
_PHASES = [list(range(p, min(p + _NBUF, len(_CHUNKS))))
           for p in range(0, len(_CHUNKS), _NBUF)]


def _phase_copy_kernel(x_ref, o_ref, buf, rsem, wsem):
    def rd(i, slot):
        return pltpu.make_async_copy(
            x_ref.at[pl.ds(_STARTS[i], _CHUNKS[i])],
            buf.at[slot, pl.ds(0, _CHUNKS[i])],
            rsem.at[slot],
        )

    def wr(i, slot):
        return pltpu.make_async_copy(
            buf.at[slot, pl.ds(0, _CHUNKS[i])],
            o_ref.at[pl.ds(_STARTS[i], _CHUNKS[i])],
            wsem.at[slot],
        )

    prev = None
    for phase in _PHASES:
        for s, i in enumerate(phase):
            if prev is not None:
                wr(prev[s], s).wait()
            rd(i, s).start()
        for s, i in enumerate(phase):
            rd(i, s).wait()
        for s, i in enumerate(phase):
            wr(i, s).start()
        prev = phase
    for s, i in enumerate(prev):
        wr(i, s).wait()


def kernel(x, expert_indices):
    del expert_indices
    rows, cols = x.shape
    return pl.pallas_call(
        _phase_copy_kernel,
        out_shape=jax.ShapeDtypeStruct(x.shape, x.dtype),
        in_specs=[pl.BlockSpec(memory_space=pl.ANY)],
        out_specs=pl.BlockSpec(memory_space=pl.ANY),
        scratch_shapes=[
            pltpu.VMEM((_NBUF, _SLOT_ROWS, cols), x.dtype),
            pltpu.SemaphoreType.DMA((_NBUF,)),
            pltpu.SemaphoreType.DMA((_NBUF,)),
        ],
    )(x)


# speedup vs baseline: 1.0244x; 1.0244x over previous
import jax
import jax.numpy as jnp
from jax.experimental import pallas as pl
from jax.experimental.pallas import tpu as pltpu

_CHUNKS = [256, 1024] + [1152] * 12 + [1024, 256]
_STARTS = [sum(_CHUNKS[:i]) for i in range(len(_CHUNKS))]
_NBUF = 3
_SLOT_ROWS = 1152
_LEAD = 2


def _pipeline_copy_kernel(x_ref, o_ref, buf, rsem, wsem):
    n = len(_CHUNKS)

    def rd(i, slot):
        return pltpu.make_async_copy(
            x_ref.at[pl.ds(_STARTS[i], _CHUNKS[i])],
            buf.at[slot, pl.ds(0, _CHUNKS[i])],
            rsem.at[slot],
        )

    def wr(i, slot):
        return pltpu.make_async_copy(
            buf.at[slot, pl.ds(0, _CHUNKS[i])],
            o_ref.at[pl.ds(_STARTS[i], _CHUNKS[i])],
            wsem.at[slot],
        )

    for i in range(n + _LEAD):
        if i < n:
            slot = i % _NBUF
            if i >= _NBUF:
                wr(i - _NBUF, slot).wait()
            rd(i, slot).start()
        if i >= _LEAD:
            j = i - _LEAD
            js = j % _NBUF
            rd(j, js).wait()
            wr(j, js).start()
    for k in range(_NBUF):
        j = n - _NBUF + k
        wr(j, j % _NBUF).wait()


def kernel(x, expert_indices):
    del expert_indices
    rows, cols = x.shape
    return pl.pallas_call(
        _pipeline_copy_kernel,
        out_shape=jax.ShapeDtypeStruct(x.shape, x.dtype),
        in_specs=[pl.BlockSpec(memory_space=pl.ANY)],
        out_specs=pl.BlockSpec(memory_space=pl.ANY),
        scratch_shapes=[
            pltpu.VMEM((_NBUF, _SLOT_ROWS, cols), x.dtype),
            pltpu.SemaphoreType.DMA((_NBUF,)),
            pltpu.SemaphoreType.DMA((_NBUF,)),
        ],
    )(x)
